# Initial kernel scaffold; baseline (speedup 1.0000x reference)
import jax
import jax.numpy as jnp
from jax.experimental import pallas as pl


def kernel(x):
    raise NotImplementedError("write your pallas kernel here")


# speedup vs baseline: 10.5635x; 10.5635x over previous
import functools

import jax
import jax.numpy as jnp
from jax import lax
from jax.experimental import pallas as pl
from jax.experimental.pallas import tpu as pltpu
from jax.experimental.pallas import tpu_sc as plsc

ROWS = 64
N = 32768
L = 16
NSLICES = N // L
WORKERS = 32
ROWS_PER_WORKER = ROWS // WORKERS


def _sparsemax_body(x_hbm, out_hbm, row_v):
    wid = lax.axis_index("s") * 2 + lax.axis_index("c")
    for r in range(ROWS_PER_WORKER):
        row = wid * ROWS_PER_WORKER + r
        pltpu.sync_copy(x_hbm.at[row], row_v)

        def max_body(i, acc):
            return jnp.maximum(acc, row_v[pl.ds(i * L, L)])

        m16 = lax.fori_loop(
            0, NSLICES, max_body, jnp.full((L,), -jnp.inf, jnp.float32)
        )
        m = lax.broadcast_in_dim(jnp.max(m16), (L,), ())

        def n_cond(carry):
            t, t_prev = carry
            return jnp.all(t > t_prev)

        def n_body(carry):
            t, _ = carry

            def pass_body(i, acc):
                sa, ca = acc
                v = row_v[pl.ds(i * L, L)]
                d = v - t
                pos = d > 0.0
                sa = sa + jnp.where(pos, d, 0.0)
                ca = ca + jnp.where(pos, 1.0, 0.0)
                return (sa, ca)

            sa, ca = lax.fori_loop(
                0,
                NSLICES,
                pass_body,
                (jnp.zeros((L,), jnp.float32), jnp.zeros((L,), jnp.float32)),
            )
            s = lax.broadcast_in_dim(jnp.sum(sa), (L,), ())
            c = lax.broadcast_in_dim(jnp.sum(ca), (L,), ())
            t_new = t + (s - 1.0) / c
            return (jnp.where(t_new > t, t_new, t), t)

        tau, _ = lax.while_loop(
            n_cond, n_body, (m - 1.0, jnp.full((L,), -jnp.inf, jnp.float32))
        )

        def out_body(i, carry):
            v = row_v[pl.ds(i * L, L)]
            row_v[pl.ds(i * L, L)] = jnp.maximum(v - tau, 0.0)
            return carry

        lax.fori_loop(0, NSLICES, out_body, 0)
        pltpu.sync_copy(row_v, out_hbm.at[row])


@jax.jit
def kernel(x):
    return pl.kernel(
        _sparsemax_body,
        out_type=jax.ShapeDtypeStruct((ROWS, N), jnp.float32),
        mesh=plsc.VectorSubcoreMesh(core_axis_name="c", subcore_axis_name="s"),
        scratch_types=[pltpu.VMEM((N,), jnp.float32)],
        compiler_params=pltpu.CompilerParams(needs_layout_passes=False),
    )(x)


# speedup vs baseline: 16.4930x; 1.5613x over previous
import functools

import jax
import jax.numpy as jnp
from jax import lax
from jax.experimental import pallas as pl
from jax.experimental.pallas import tpu as pltpu
from jax.experimental.pallas import tpu_sc as plsc

ROWS = 64
N = 32768
L = 16
NSLICES = N // L
WORKERS = 32
ROWS_PER_WORKER = ROWS // WORKERS
NEG = -3.0e38


def _sparsemax_body(x_hbm, out_hbm, row_v, cand_v):
    wid = lax.axis_index("s") * 2 + lax.axis_index("c")
    for r in range(ROWS_PER_WORKER):
        row = wid * ROWS_PER_WORKER + r
        pltpu.sync_copy(x_hbm.at[row], row_v)

        def cpt_body(i, carry):
            off, w = carry
            v = row_v[pl.ds(i * L, L)]
            pos = v > w
            plsc.store_compressed(cand_v.at[pl.ds(off, L)], v, mask=pos)
            cnt = plsc.all_reduce_population_count(pos)[0]
            w = jnp.maximum(w, v - 1.0)
            return (off + cnt, w)

        k_count, _ = lax.fori_loop(
            0,
            NSLICES,
            cpt_body,
            (jnp.int32(0), jnp.full((L,), NEG, jnp.float32)),
        )
        cand_v[pl.ds(k_count, L)] = jnp.full((L,), NEG, jnp.float32)
        nsl = (k_count + (L - 1)) >> 4

        def max_body(i, acc):
            return jnp.maximum(acc, cand_v[pl.ds(i * L, L)])

        m16 = lax.fori_loop(
            0, nsl, max_body, jnp.full((L,), NEG, jnp.float32)
        )
        m = lax.broadcast_in_dim(jnp.max(m16), (L,), ())

        def n_cond(carry):
            t, t_prev = carry
            return jnp.all(t > t_prev)

        def n_body(carry):
            t, _ = carry

            def pass_body(i, acc):
                sa, ca = acc
                v = cand_v[pl.ds(i * L, L)]
                d = v - t
                pos = d > 0.0
                sa = sa + jnp.where(pos, d, 0.0)
                ca = ca + jnp.where(pos, 1.0, 0.0)
                return (sa, ca)

            sa, ca = lax.fori_loop(
                0,
                nsl,
                pass_body,
                (jnp.zeros((L,), jnp.float32), jnp.zeros((L,), jnp.float32)),
            )
            s = lax.broadcast_in_dim(jnp.sum(sa), (L,), ())
            c = lax.broadcast_in_dim(jnp.sum(ca), (L,), ())
            t_new = t + (s - 1.0) / c
            return (jnp.where(t_new > t, t_new, t), t)

        tau, _ = lax.while_loop(
            n_cond, n_body, (m - 1.0, jnp.full((L,), NEG, jnp.float32))
        )

        def out_body(i, carry):
            v = row_v[pl.ds(i * L, L)]
            row_v[pl.ds(i * L, L)] = jnp.maximum(v - tau, 0.0)
            return carry

        lax.fori_loop(0, NSLICES, out_body, 0)
        pltpu.sync_copy(row_v, out_hbm.at[row])


@jax.jit
def kernel(x):
    return pl.kernel(
        _sparsemax_body,
        out_type=jax.ShapeDtypeStruct((ROWS, N), jnp.float32),
        mesh=plsc.VectorSubcoreMesh(core_axis_name="c", subcore_axis_name="s"),
        scratch_types=[
            pltpu.VMEM((N,), jnp.float32),
            pltpu.VMEM((N + L,), jnp.float32),
        ],
        compiler_params=pltpu.CompilerParams(needs_layout_passes=False),
    )(x)


# speedup vs baseline: 37.5226x; 2.2751x over previous
import functools

import jax
import jax.numpy as jnp
from jax import lax
from jax.experimental import pallas as pl
from jax.experimental.pallas import tpu as pltpu
from jax.experimental.pallas import tpu_sc as plsc

ROWS = 64
N = 32768
L = 16
NSLICES = N // L
WORKERS = 32
ROWS_PER_WORKER = ROWS // WORKERS
NEG = -3.0e38


def _sparsemax_body(x_hbm, out_hbm, row_v, cand_v):
    wid = lax.axis_index("s") * 2 + lax.axis_index("c")
    for r in range(ROWS_PER_WORKER):
        row = wid * ROWS_PER_WORKER + r
        pltpu.sync_copy(x_hbm.at[row], row_v)

        U = 8

        def cpt_body(i, carry):
            off, w = carry
            vs = [row_v[pl.ds((i * U + u) * L, L)] for u in range(U)]
            ps = [v > w for v in vs]
            cs = [plsc.all_reduce_population_count(p)[0] for p in ps]
            for u in range(U):
                plsc.store_compressed(
                    cand_v.at[pl.ds(off, L)], vs[u], mask=ps[u]
                )
                off = off + cs[u]
            wa = jnp.maximum(jnp.maximum(vs[0], vs[1]),
                             jnp.maximum(vs[2], vs[3]))
            wb = jnp.maximum(jnp.maximum(vs[4], vs[5]),
                             jnp.maximum(vs[6], vs[7]))
            w = jnp.maximum(w, jnp.maximum(wa, wb) - 1.0)
            return (off, w)

        k_count, _ = lax.fori_loop(
            0,
            NSLICES // U,
            cpt_body,
            (jnp.int32(0), jnp.full((L,), NEG, jnp.float32)),
            unroll=1,
        )
        cand_v[pl.ds(k_count, L)] = jnp.full((L,), NEG, jnp.float32)
        nsl = (k_count + (L - 1)) >> 4

        def max_body(i, acc):
            return jnp.maximum(acc, cand_v[pl.ds(i * L, L)])

        m16 = lax.fori_loop(
            0, nsl, max_body, jnp.full((L,), NEG, jnp.float32)
        )
        m = lax.broadcast_in_dim(jnp.max(m16), (L,), ())

        def n_cond(carry):
            t, t_prev = carry
            return jnp.all(t > t_prev)

        def n_body(carry):
            t, _ = carry

            def pass_body(i, acc):
                sa, ca = acc
                v = cand_v[pl.ds(i * L, L)]
                d = v - t
                pos = d > 0.0
                sa = sa + jnp.where(pos, d, 0.0)
                ca = ca + jnp.where(pos, 1.0, 0.0)
                return (sa, ca)

            sa, ca = lax.fori_loop(
                0,
                nsl,
                pass_body,
                (jnp.zeros((L,), jnp.float32), jnp.zeros((L,), jnp.float32)),
            )
            s = lax.broadcast_in_dim(jnp.sum(sa), (L,), ())
            c = lax.broadcast_in_dim(jnp.sum(ca), (L,), ())
            t_new = t + (s - 1.0) / c
            return (jnp.where(t_new > t, t_new, t), t)

        tau, _ = lax.while_loop(
            n_cond, n_body, (m - 1.0, jnp.full((L,), NEG, jnp.float32))
        )

        def out_body(i, carry):
            for u in range(U):
                sl = pl.ds((i * U + u) * L, L)
                row_v[sl] = jnp.maximum(row_v[sl] - tau, 0.0)
            return carry

        lax.fori_loop(0, NSLICES // U, out_body, 0, unroll=1)
        pltpu.sync_copy(row_v, out_hbm.at[row])


@jax.jit
def kernel(x):
    return pl.kernel(
        _sparsemax_body,
        out_type=jax.ShapeDtypeStruct((ROWS, N), jnp.float32),
        mesh=plsc.VectorSubcoreMesh(core_axis_name="c", subcore_axis_name="s"),
        scratch_types=[
            pltpu.VMEM((N,), jnp.float32),
            pltpu.VMEM((N + L,), jnp.float32),
        ],
        compiler_params=pltpu.CompilerParams(needs_layout_passes=False),
    )(x)


# speedup vs baseline: 37.8178x; 1.0079x over previous
import functools

import jax
import jax.numpy as jnp
from jax import lax
from jax.experimental import pallas as pl
from jax.experimental.pallas import tpu as pltpu
from jax.experimental.pallas import tpu_sc as plsc

ROWS = 64
N = 32768
L = 16
NSLICES = N // L
WORKERS = 32
ROWS_PER_WORKER = ROWS // WORKERS
NEG = -3.0e38


def _sparsemax_body(x_hbm, out_hbm, row_v, cand_v):
    wid = lax.axis_index("s") * 2 + lax.axis_index("c")

    def row_body(r, row_carry):
        row = wid * ROWS_PER_WORKER + r
        pltpu.sync_copy(x_hbm.at[row], row_v)

        U = 8

        def cpt_body(i, carry):
            off, w = carry
            vs = [row_v[pl.ds((i * U + u) * L, L)] for u in range(U)]
            ps = [v > w for v in vs]
            cs = [plsc.all_reduce_population_count(p)[0] for p in ps]
            for u in range(U):
                plsc.store_compressed(
                    cand_v.at[pl.ds(off, L)], vs[u], mask=ps[u]
                )
                off = off + cs[u]
            wa = jnp.maximum(jnp.maximum(vs[0], vs[1]),
                             jnp.maximum(vs[2], vs[3]))
            wb = jnp.maximum(jnp.maximum(vs[4], vs[5]),
                             jnp.maximum(vs[6], vs[7]))
            w = jnp.maximum(w, jnp.maximum(wa, wb) - 1.0)
            return (off, w)

        k_count, _ = lax.fori_loop(
            0,
            NSLICES // U,
            cpt_body,
            (jnp.int32(0), jnp.full((L,), NEG, jnp.float32)),
            unroll=1,
        )
        cand_v[pl.ds(k_count, L)] = jnp.full((L,), NEG, jnp.float32)
        nsl = (k_count + (L - 1)) >> 4

        def max_body(i, acc):
            return jnp.maximum(acc, cand_v[pl.ds(i * L, L)])

        m16 = lax.fori_loop(
            0, nsl, max_body, jnp.full((L,), NEG, jnp.float32)
        )
        m = lax.broadcast_in_dim(jnp.max(m16), (L,), ())

        def n_cond(carry):
            t, t_prev = carry
            return jnp.all(t > t_prev)

        def n_body(carry):
            t, _ = carry

            def pass_body(i, acc):
                sa, ca = acc
                v = cand_v[pl.ds(i * L, L)]
                d = v - t
                pos = d > 0.0
                sa = sa + jnp.where(pos, d, 0.0)
                ca = ca + jnp.where(pos, 1.0, 0.0)
                return (sa, ca)

            sa, ca = lax.fori_loop(
                0,
                nsl,
                pass_body,
                (jnp.zeros((L,), jnp.float32), jnp.zeros((L,), jnp.float32)),
            )
            s = lax.broadcast_in_dim(jnp.sum(sa), (L,), ())
            c = lax.broadcast_in_dim(jnp.sum(ca), (L,), ())
            t_new = t + (s - 1.0) / c
            return (jnp.where(t_new > t, t_new, t), t)

        tau, _ = lax.while_loop(
            n_cond, n_body, (m - 1.0, jnp.full((L,), NEG, jnp.float32))
        )

        def out_body(i, carry):
            for u in range(U):
                sl = pl.ds((i * U + u) * L, L)
                row_v[sl] = jnp.maximum(row_v[sl] - tau, 0.0)
            return carry

        lax.fori_loop(0, NSLICES // U, out_body, 0, unroll=1)
        pltpu.sync_copy(row_v, out_hbm.at[row])
        return row_carry

    lax.fori_loop(0, ROWS_PER_WORKER, row_body, 0)


@jax.jit
def kernel(x):
    return pl.kernel(
        _sparsemax_body,
        out_type=jax.ShapeDtypeStruct((ROWS, N), jnp.float32),
        mesh=plsc.VectorSubcoreMesh(core_axis_name="c", subcore_axis_name="s"),
        scratch_types=[
            pltpu.VMEM((N,), jnp.float32),
            pltpu.VMEM((N + L,), jnp.float32),
        ],
        compiler_params=pltpu.CompilerParams(needs_layout_passes=False),
    )(x)


# speedup vs baseline: 42.0737x; 1.1125x over previous
import functools

import jax
import jax.numpy as jnp
from jax import lax
from jax.experimental import pallas as pl
from jax.experimental.pallas import tpu as pltpu
from jax.experimental.pallas import tpu_sc as plsc

ROWS = 64
N = 32768
L = 16
NSLICES = N // L
WORKERS = 32
ROWS_PER_WORKER = ROWS // WORKERS
NEG = -3.0e38
U = 8


def _process_row(buf, cand_v):
    def cpt_body(i, carry):
        offm1, w = carry
        vs = [buf[pl.ds((i * U + u) * L, L)] for u in range(U)]
        ps = [v > w for v in vs]
        for u in range(U):
            pc = plsc.cumsum(jnp.where(ps[u], 1, 0))
            plsc.store_scatter(cand_v, [offm1 + pc], vs[u], mask=ps[u])
            offm1 = offm1 + plsc.all_reduce_population_count(ps[u])
        wa = jnp.maximum(jnp.maximum(vs[0], vs[1]),
                         jnp.maximum(vs[2], vs[3]))
        wb = jnp.maximum(jnp.maximum(vs[4], vs[5]),
                         jnp.maximum(vs[6], vs[7]))
        w = jnp.maximum(w, jnp.maximum(wa, wb) - 1.0)
        return (offm1, w)

    offm1, _ = lax.fori_loop(
        0,
        NSLICES // U,
        cpt_body,
        (jnp.full((L,), -1, jnp.int32), jnp.full((L,), NEG, jnp.float32)),
        unroll=1,
    )
    k_count = jnp.max(offm1) + 1
    cand_v[pl.ds(k_count, L)] = jnp.full((L,), NEG, jnp.float32)
    nsl = (k_count + (L - 1)) >> 4

    def max_body(i, acc):
        return jnp.maximum(acc, cand_v[pl.ds(i * L, L)])

    m16 = lax.fori_loop(0, nsl, max_body, jnp.full((L,), NEG, jnp.float32))
    m = lax.broadcast_in_dim(jnp.max(m16), (L,), ())

    def n_cond(carry):
        t, t_prev = carry
        return jnp.all(t > t_prev)

    def n_body(carry):
        t, _ = carry

        def pass_body(i, acc):
            sa, ca = acc
            v = cand_v[pl.ds(i * L, L)]
            d = v - t
            sa = sa + jnp.maximum(d, 0.0)
            ca = ca + plsc.all_reduce_population_count(d > 0.0)
            return (sa, ca)

        sa, ca = lax.fori_loop(
            0,
            nsl,
            pass_body,
            (jnp.zeros((L,), jnp.float32), jnp.zeros((L,), jnp.int32)),
        )
        s = lax.broadcast_in_dim(jnp.sum(sa), (L,), ())
        c = ca.astype(jnp.float32)
        t_new = t + (s - 1.0) / c
        return (jnp.where(t_new > t, t_new, t), t)

    tau, _ = lax.while_loop(
        n_cond, n_body, (m - 1.0, jnp.full((L,), NEG, jnp.float32))
    )

    def out_body(i, carry):
        for u in range(U):
            sl = pl.ds((i * U + u) * L, L)
            buf[sl] = jnp.maximum(buf[sl] - tau, 0.0)
        return carry

    lax.fori_loop(0, NSLICES // U, out_body, 0, unroll=1)


def _sparsemax_body(x_hbm, out_hbm, buf_a, buf_b, cand_v, sem_a, sem_b,
                    sem_oa, sem_ob):
    wid = lax.axis_index("s") * 2 + lax.axis_index("c")
    row0 = wid * ROWS_PER_WORKER
    row1 = row0 + 1
    in_a = pltpu.async_copy(x_hbm.at[row0], buf_a, sem_a)
    in_b = pltpu.async_copy(x_hbm.at[row1], buf_b, sem_b)
    in_a.wait()
    _process_row(buf_a, cand_v)
    out_a = pltpu.async_copy(buf_a, out_hbm.at[row0], sem_oa)
    in_b.wait()
    _process_row(buf_b, cand_v)
    out_b = pltpu.async_copy(buf_b, out_hbm.at[row1], sem_ob)
    out_a.wait()
    out_b.wait()


@jax.jit
def kernel(x):
    return pl.kernel(
        _sparsemax_body,
        out_type=jax.ShapeDtypeStruct((ROWS, N), jnp.float32),
        mesh=plsc.VectorSubcoreMesh(core_axis_name="c", subcore_axis_name="s"),
        scratch_types=[
            pltpu.VMEM((N,), jnp.float32),
            pltpu.VMEM((N,), jnp.float32),
            pltpu.VMEM((N + L,), jnp.float32),
            pltpu.SemaphoreType.DMA,
            pltpu.SemaphoreType.DMA,
            pltpu.SemaphoreType.DMA,
            pltpu.SemaphoreType.DMA,
        ],
        compiler_params=pltpu.CompilerParams(needs_layout_passes=False),
    )(x)
